# baseline (speedup 1.0000x reference)
import functools

import jax
import jax.numpy as jnp
from jax import lax
from jax.experimental import pallas as pl
from jax.experimental.pallas import tpu as pltpu
from jax.experimental.pallas import tpu_sc as plsc

NC = 2
NS = 16
NW = NC * NS

BATCH = 16384
HIST = 50
DIM = 64
LB = 128
NTB = BATCH // LB
TB_PER_W = NTB // NW
NSLOT = 2


@functools.lru_cache(maxsize=None)
def _build(vhalf: int):
    mesh = plsc.VectorSubcoreMesh(
        core_axis_name="c", subcore_axis_name="s",
        num_cores=NC, num_subcores=NS)

    @functools.partial(
        pl.kernel,
        out_type=jax.ShapeDtypeStruct((HIST, DIM, BATCH), jnp.float32),
        mesh=mesh,
        compiler_params=pltpu.CompilerParams(use_tc_tiling_on_sc=True,
                                             needs_layout_passes=False),
        scratch_types=[
            pltpu.VMEM((TB_PER_W * HIST, LB), jnp.int32),
            [pltpu.VMEM((LB,), jnp.int32)] * NSLOT,
            [pltpu.VMEM((LB,), jnp.int32)] * NSLOT,
            [pltpu.VMEM((LB, LB), jnp.float32)] * NSLOT,
            [pltpu.VMEM((DIM, LB), jnp.float32)] * NSLOT,
            pltpu.SemaphoreType.DMA((NSLOT,)),
            pltpu.SemaphoreType.DMA((NSLOT,)),
        ],
    )
    def emb(x2, t2, out, xv, gidx, sidev, rows, tbuf, gsem, osem):
        wid = lax.axis_index("s") * NC + lax.axis_index("c")
        iota = lax.iota(jnp.int32, 16)

        def prep(toff, h, s):
            for g in range(8):
                v = iota * HIST + (toff + g * 16 * HIST + h)
                xval = plsc.load_gather(xv, [v >> 7, v & 127])
                gidx[s][pl.ds(g * 16, 16)] = xval >> 1
                sidev[s][pl.ds(g * 16, 16)] = (xval & 1) * DIM

        def start_gather(s):
            pltpu.async_copy(t2.at[gidx[s]], rows[s], gsem.at[s])

        def wait_gather(s):
            pltpu.make_async_copy(t2.at[gidx[s]], rows[s], gsem.at[s]).wait()

        def transpose(s):
            for g in range(8):
                ri = iota + g * 16
                cv = sidev[s][pl.ds(g * 16, 16)]

                @plsc.parallel_loop(0, DIM, step=1, unroll=16)
                def _(d):
                    vals = plsc.load_gather(rows[s], [ri, cv + d])
                    tbuf[s][d, pl.ds(g * 16, 16)] = vals

        def start_out(tbg, h, s):
            pltpu.async_copy(tbuf[s],
                             out.at[h, :, pl.ds(tbg * LB, LB)], osem.at[s])

        def wait_out(tbg, h, s):
            pltpu.make_async_copy(tbuf[s],
                                  out.at[h, :, pl.ds(tbg * LB, LB)],
                                  osem.at[s]).wait()

        pltpu.sync_copy(
            x2.at[pl.ds(wid * (TB_PER_W * HIST), TB_PER_W * HIST), :], xv)

        def tb_body(t, carry):
            tbg = wid * TB_PER_W + t
            toff = t * (HIST * LB)
            for s in range(NSLOT):
                prep(toff, s, s)
                start_gather(s)

            def pair_body(i, carry2):
                for s in range(NSLOT):
                    h = i * NSLOT + s
                    wait_gather(s)

                    @pl.when(h >= NSLOT)
                    def _():
                        wait_out(tbg, h - NSLOT, s)

                    transpose(s)
                    start_out(tbg, h, s)

                    @pl.when(h + NSLOT < HIST)
                    def _():
                        prep(toff, h + NSLOT, s)
                        start_gather(s)
                return carry2

            lax.fori_loop(0, HIST // NSLOT, pair_body, 0)
            for s in range(NSLOT):
                wait_out(tbg, HIST - NSLOT + s, s)
            return carry

        lax.fori_loop(0, TB_PER_W, tb_body, 0)

    return emb


def kernel(x, embeddings):
    V, D = embeddings.shape
    x2 = x.reshape(BATCH * HIST // 128, 128).astype(jnp.int32)
    t2 = embeddings.reshape(V * D // 128, 128)
    outp = _build(V * D // 128)(x2, t2)
    return jnp.transpose(outp, (2, 0, 1))


# speedup vs baseline: 1.0481x; 1.0481x over previous
import functools

import jax
import jax.numpy as jnp
from jax import lax
from jax.experimental import pallas as pl
from jax.experimental.pallas import tpu as pltpu
from jax.experimental.pallas import tpu_sc as plsc

NC = 2
NS = 16
NW = NC * NS

BATCH = 16384
HIST = 50
DIM = 64
LB = 128
NTB = BATCH // LB
TB_PER_W = NTB // NW
NSLOT = 2


@functools.lru_cache(maxsize=None)
def _build(vhalf: int):
    mesh = plsc.VectorSubcoreMesh(
        core_axis_name="c", subcore_axis_name="s",
        num_cores=NC, num_subcores=NS)

    @functools.partial(
        pl.kernel,
        out_type=jax.ShapeDtypeStruct((HIST, DIM, BATCH), jnp.float32),
        mesh=mesh,
        compiler_params=pltpu.CompilerParams(use_tc_tiling_on_sc=True,
                                             needs_layout_passes=False),
        scratch_types=[
            pltpu.VMEM((TB_PER_W * HIST, LB), jnp.int32),
            [pltpu.VMEM((LB,), jnp.int32)] * NSLOT,
            [pltpu.VMEM((LB, LB), jnp.float32)] * NSLOT,
            [pltpu.VMEM((DIM, LB), jnp.float32)] * NSLOT,
            pltpu.SemaphoreType.DMA((NSLOT,)),
            pltpu.SemaphoreType.DMA((NSLOT,)),
        ],
    )
    def emb(x2, t2, out, xv, gidx, rows, tbuf, gsem, osem):
        wid = lax.axis_index("s") * NC + lax.axis_index("c")
        iota = lax.iota(jnp.int32, 16)

        def prep(toff, h, s):
            for g in range(8):
                v = iota * HIST + (toff + g * 16 * HIST + h)
                xval = plsc.load_gather(xv, [v >> 7, v & 127])
                gidx[s][pl.ds(g * 16, 16)] = xval

        def start_gather(s):
            pltpu.async_copy(t2.at[gidx[s]], rows[s], gsem.at[s])

        def wait_gather(s):
            pltpu.make_async_copy(t2.at[gidx[s]], rows[s], gsem.at[s]).wait()

        def transpose(s):
            zero = iota * 0
            for g in range(8):
                ri = iota + g * 16

                @plsc.parallel_loop(0, DIM, step=1, unroll=16)
                def _(d):
                    vals = plsc.load_gather(rows[s], [ri, zero + d])
                    tbuf[s][d, pl.ds(g * 16, 16)] = vals

        def start_out(tbg, h, s):
            pltpu.async_copy(tbuf[s],
                             out.at[h, :, pl.ds(tbg * LB, LB)], osem.at[s])

        def wait_out(tbg, h, s):
            pltpu.make_async_copy(tbuf[s],
                                  out.at[h, :, pl.ds(tbg * LB, LB)],
                                  osem.at[s]).wait()

        pltpu.sync_copy(
            x2.at[pl.ds(wid * (TB_PER_W * HIST), TB_PER_W * HIST), :], xv)

        def tb_body(t, carry):
            tbg = wid * TB_PER_W + t
            toff = t * (HIST * LB)
            for s in range(NSLOT):
                prep(toff, s, s)
                start_gather(s)

            def pair_body(i, carry2):
                for s in range(NSLOT):
                    h = i * NSLOT + s
                    wait_gather(s)

                    @pl.when(h >= NSLOT)
                    def _():
                        wait_out(tbg, h - NSLOT, s)

                    transpose(s)
                    start_out(tbg, h, s)

                    @pl.when(h + NSLOT < HIST)
                    def _():
                        prep(toff, h + NSLOT, s)
                        start_gather(s)
                return carry2

            lax.fori_loop(0, HIST // NSLOT, pair_body, 0)
            for s in range(NSLOT):
                wait_out(tbg, HIST - NSLOT + s, s)
            return carry

        lax.fori_loop(0, TB_PER_W, tb_body, 0)

    return emb


def kernel(x, embeddings):
    V, D = embeddings.shape
    x2 = x.reshape(BATCH * HIST // 128, 128).astype(jnp.int32)
    t2 = jnp.pad(embeddings, ((0, 0), (0, 128 - D)))
    outp = _build(V)(x2, t2)
    return jnp.transpose(outp, (2, 0, 1))


# speedup vs baseline: 1.0489x; 1.0008x over previous
import functools

import jax
import jax.numpy as jnp
from jax import lax
from jax.experimental import pallas as pl
from jax.experimental.pallas import tpu as pltpu
from jax.experimental.pallas import tpu_sc as plsc

NC = 2
NS = 16
NW = NC * NS

BATCH = 16384
HIST = 50
DIM = 64
LB = 128
NTB = BATCH // LB
TB_PER_W = NTB // NW
NSLOT = 2


@functools.lru_cache(maxsize=None)
def _build(vocab: int):
    mesh = plsc.VectorSubcoreMesh(
        core_axis_name="c", subcore_axis_name="s",
        num_cores=NC, num_subcores=NS)

    @functools.partial(
        pl.kernel,
        out_type=jax.ShapeDtypeStruct((HIST, DIM, BATCH), jnp.float32),
        mesh=mesh,
        compiler_params=pltpu.CompilerParams(use_tc_tiling_on_sc=True,
                                             needs_layout_passes=False),
        scratch_types=[
            pltpu.VMEM((TB_PER_W * HIST, LB), jnp.int32),
            [pltpu.VMEM((LB,), jnp.int32)] * NSLOT,
            [pltpu.VMEM((LB, LB), jnp.float32)] * NSLOT,
            [pltpu.VMEM((DIM, LB), jnp.float32)] * NSLOT,
            pltpu.SemaphoreType.DMA((NSLOT,)),
            pltpu.SemaphoreType.DMA((NSLOT,)),
        ],
    )
    def emb(x2, t2, out, xv, gidx, rows, tbuf, gsem, osem):
        wid = lax.axis_index("s") * NC + lax.axis_index("c")
        iota = lax.iota(jnp.int32, 16)

        def prep(toff, h, s):
            for g in range(8):
                v = iota * HIST + (toff + g * 16 * HIST + h)
                xval = plsc.load_gather(xv, [v >> 7, v & 127])
                gidx[s][pl.ds(g * 16, 16)] = xval

        def start_gather(s):
            pltpu.async_copy(t2.at[gidx[s]], rows[s], gsem.at[s])

        def wait_gather(s):
            pltpu.make_async_copy(t2.at[gidx[s]], rows[s], gsem.at[s]).wait()

        def transpose(s):
            zero = iota * 0
            for g in range(8):
                ri = iota + g * 16

                @plsc.parallel_loop(0, DIM, step=1, unroll=16)
                def _(d):
                    vals = plsc.load_gather(rows[s], [ri, zero + d])
                    tbuf[s][d, pl.ds(g * 16, 16)] = vals

        def start_out(tbg, h, s):
            pltpu.async_copy(tbuf[s],
                             out.at[h, :, pl.ds(tbg * LB, LB)], osem.at[s])

        def wait_out(tbg, h, s):
            pltpu.make_async_copy(tbuf[s],
                                  out.at[h, :, pl.ds(tbg * LB, LB)],
                                  osem.at[s]).wait()

        pltpu.sync_copy(
            x2.at[pl.ds(wid * (TB_PER_W * HIST), TB_PER_W * HIST), :], xv)

        def tb_body(t, carry):
            tbg = wid * TB_PER_W + t
            toff = t * (HIST * LB)
            for s in range(NSLOT):
                prep(toff, s, s)
                start_gather(s)

            def pair_body(i, carry2):
                for s in range(NSLOT):
                    h = i * NSLOT + s
                    wait_gather(s)

                    @pl.when(h >= NSLOT)
                    def _():
                        wait_out(tbg, h - NSLOT, s)

                    transpose(s)
                    start_out(tbg, h, s)

                    @pl.when(h + NSLOT < HIST)
                    def _():
                        prep(toff, h + NSLOT, s)
                        start_gather(s)
                return carry2

            lax.fori_loop(0, HIST // NSLOT, pair_body, 0)
            for s in range(NSLOT):
                wait_out(tbg, HIST - NSLOT + s, s)
            return carry

        lax.fori_loop(0, TB_PER_W, tb_body, 0)

    return emb


def kernel(x, embeddings):
    V, D = embeddings.shape
    x2 = x.reshape(BATCH * HIST // 128, 128).astype(jnp.int32)
    t2 = jnp.pad(embeddings, ((0, 0), (0, 128 - D)))
    outp = _build(V)(x2, t2)
    return jnp.transpose(outp, (2, 0, 1))
